# baseline (device time: 99599 ns/iter reference)
import jax
import jax.numpy as jnp
from jax import lax
from jax.experimental import pallas as pl
from jax.experimental.pallas import tpu as pltpu

N_DEV = 4
B, S, D = 2, 512, 2048
H, DH, DR = 16, 128, 32
DC = 512
DCS = DC // N_DEV
BS = B * S
N_COMM = 3
N_PEER = N_DEV - 1
NO = 8
DO = D // NO

BF = jnp.bfloat16
F32 = jnp.float32


def _gather_body(x_ref, wdkv_ref, wuk_ref, wuv_ref, wkr_ref, wqr_ref, wq_ref,
                 q_ref, k_ref, v_ref, kr_ref, qr_ref,
                 c_ref, wukf_ref, wuvf_ref, xbf_ref, wqh_ref,
                 p1_send, p1_recv, p2_send, p2_recv, copy_sem):
    my = lax.axis_index("i")
    right = lax.rem(my + 1, N_DEV)
    left = lax.rem(my + N_DEV - 1, N_DEV)

    for b in range(B):
        xbf_ref[b * S:(b + 1) * S, :] = x_ref[b].astype(BF)
    xbf = xbf_ref[...]
    c_ref[my] = jnp.dot(xbf, wdkv_ref[...].astype(BF),
                        preferred_element_type=F32).astype(BF)
    wukf_ref[my] = wuk_ref[...].astype(BF)
    wuvf_ref[my] = wuv_ref[...].astype(BF)

    tensors = (c_ref, wukf_ref, wuvf_ref)
    halves = (BS // 2, DCS // 2, DCS // 2)
    p1 = []
    for d, dst in enumerate((right, left)):
        for t, ref in enumerate(tensors):
            rdma = pltpu.make_async_remote_copy(
                src_ref=ref.at[my],
                dst_ref=ref.at[my],
                send_sem=p1_send.at[d, t],
                recv_sem=p1_recv.at[d, t],
                device_id=(dst,),
                device_id_type=pl.DeviceIdType.MESH,
            )
            rdma.start()
            p1.append(rdma)

    DQ = D // 2
    for j in range(2):
        cp = pltpu.make_async_copy(
            wq_ref.at[:, pl.ds(j * DQ, DQ)], wqh_ref, copy_sem)
        cp.start()
        cp.wait()
        q_ref[:, j * DQ:(j + 1) * DQ] = jnp.dot(
            xbf, wqh_ref[...].astype(BF),
            preferred_element_type=F32).astype(BF)
    kr_ref[...] = jnp.dot(xbf, wkr_ref[...].astype(BF),
                          preferred_element_type=F32).astype(BF)
    qr_full = jnp.dot(xbf, wqr_ref[...].astype(BF),
                      preferred_element_type=F32).astype(BF)
    for hh in range(H):
        qr_ref[hh] = qr_full[:, hh * DR:(hh + 1) * DR]

    for rdma in p1:
        rdma.wait_recv()

    p2 = []
    for d, (dst, org) in enumerate(((right, left), (left, right))):
        for t, ref in enumerate(tensors):
            hs = halves[t]
            rows = pl.ds(d * hs, hs)
            rdma = pltpu.make_async_remote_copy(
                src_ref=ref.at[org, rows],
                dst_ref=ref.at[org, rows],
                send_sem=p2_send.at[d, t],
                recv_sem=p2_recv.at[d, t],
                device_id=(dst,),
                device_id_type=pl.DeviceIdType.MESH,
            )
            rdma.start()
            p2.append(rdma)
    for rdma in p2:
        rdma.wait_recv()
    for rdma in p1 + p2:
        rdma.wait_send()

    for j in range(2):
        sl = slice(j * DQ, (j + 1) * DQ)
        for src, dst in ((wukf_ref, k_ref), (wuvf_ref, v_ref)):
            acc = jnp.zeros((BS, DQ), F32)
            for o in range(N_DEV):
                acc = acc + jnp.dot(c_ref[o], src[o, :, sl],
                                    preferred_element_type=F32)
            dst[:, sl] = acc.astype(BF)


def _attn_body(q_ref, k_ref, v_ref, kr_ref, qr_h_ref, o_ref):
    scale = (DH + DR) ** -0.5
    qh = jnp.concatenate([q_ref[...], qr_h_ref[0]], axis=1)
    kh = jnp.concatenate([k_ref[...], kr_ref[...]], axis=1)
    nt = (((1,), (1,)), ((), ()))
    for b in range(B):
        sl = slice(b * S, (b + 1) * S)
        s = lax.dot_general(qh[sl], kh[sl], nt, preferred_element_type=F32)
        p = jnp.exp(s * scale)
        denom = jnp.sum(p, axis=-1, keepdims=True)
        o_b = jnp.dot(p.astype(BF), v_ref[sl], preferred_element_type=F32)
        o_ref[sl, :] = (o_b * (1.0 / denom)).astype(BF)


def _proj_body(o_ref, wo_ref, out_ref):
    out_ref[...] = jnp.dot(
        o_ref[...], wo_ref[...].astype(BF),
        preferred_element_type=F32).reshape(B, S, DO)


def kernel(x, Wdkv, Wuk, Wuv, Wq, Wqr, Wkr, Wo):
    q, k, v, kr, qr = pl.pallas_call(
        _gather_body,
        out_shape=(
            jax.ShapeDtypeStruct((BS, D), BF),
            jax.ShapeDtypeStruct((BS, D), BF),
            jax.ShapeDtypeStruct((BS, D), BF),
            jax.ShapeDtypeStruct((BS, DR), BF),
            jax.ShapeDtypeStruct((H, BS, DR), BF),
        ),
        in_specs=[pl.BlockSpec(memory_space=pltpu.VMEM)] * 6
        + [pl.BlockSpec(memory_space=pl.ANY)],
        out_specs=(pl.BlockSpec(memory_space=pltpu.VMEM),) * 5,
        scratch_shapes=[
            pltpu.VMEM((N_DEV, BS, DCS), BF),
            pltpu.VMEM((N_DEV, DCS, D), BF),
            pltpu.VMEM((N_DEV, DCS, D), BF),
            pltpu.VMEM((BS, D), BF),
            pltpu.VMEM((D, D // 2), F32),
            pltpu.SemaphoreType.DMA((2, N_COMM)),
            pltpu.SemaphoreType.DMA((2, N_COMM)),
            pltpu.SemaphoreType.DMA((2, N_COMM)),
            pltpu.SemaphoreType.DMA((2, N_COMM)),
            pltpu.SemaphoreType.DMA,
        ],
    )(x, Wdkv, Wuk, Wuv, Wkr, Wqr, Wq)

    o_all = pl.pallas_call(
        _attn_body,
        grid=(H,),
        out_shape=jax.ShapeDtypeStruct((BS, D), BF),
        in_specs=[
            pl.BlockSpec((BS, DH), lambda h: (0, h)),
            pl.BlockSpec((BS, DH), lambda h: (0, h)),
            pl.BlockSpec((BS, DH), lambda h: (0, h)),
            pl.BlockSpec((BS, DR), lambda h: (0, 0)),
            pl.BlockSpec((1, BS, DR), lambda h: (h, 0, 0)),
        ],
        out_specs=pl.BlockSpec((BS, DH), lambda h: (0, h)),
        compiler_params=pltpu.CompilerParams(
            dimension_semantics=("arbitrary",),
        ),
    )(q, k, v, kr, qr)

    out = pl.pallas_call(
        _proj_body,
        grid=(NO,),
        out_shape=jax.ShapeDtypeStruct((B, S, D), F32),
        in_specs=[
            pl.BlockSpec((BS, D), lambda n: (0, 0)),
            pl.BlockSpec((D, DO), lambda n: (0, n)),
        ],
        out_specs=pl.BlockSpec((B, S, DO), lambda n: (0, 0, n)),
        compiler_params=pltpu.CompilerParams(
            dimension_semantics=("arbitrary",),
        ),
    )(o_all, Wo)
    return out


# device time: 73201 ns/iter; 1.3606x vs baseline; 1.3606x over previous
import jax
import jax.numpy as jnp
from jax import lax
from jax.experimental import pallas as pl
from jax.experimental.pallas import tpu as pltpu

N_DEV = 4
B, S, D = 2, 512, 2048
H, DH, DR = 16, 128, 32
DC = 512
DCS = DC // N_DEV
BS = B * S
HG = H // N_DEV
DG = HG * DH
N_COMM = 3

BF = jnp.bfloat16
F32 = jnp.float32


def _fused_body(x_ref, wdkv_ref, wuk_ref, wuv_ref, wkr_ref, wqr_ref,
                wq_any, wo_any, out_ref,
                xbf_ref, c_ref, wukbf_ref, wuvbf_ref, wuk_sl, wuv_sl,
                obuf_ref, wq_st, wo_st,
                p1_send, p1_recv, o_send, o_recv, wq_sem, wo_sem):
    my = lax.axis_index("i")

    wq_cp = pltpu.make_async_copy(
        wq_any.at[:, pl.ds(my * DG, DG)], wq_st, wq_sem)
    wq_cp.start()
    wo_cp = pltpu.make_async_copy(
        wo_any.at[pl.ds(my * DG, DG), :], wo_st, wo_sem)
    wo_cp.start()

    for b in range(B):
        xbf_ref[b * S:(b + 1) * S, :] = x_ref[b].astype(BF)
    xbf = xbf_ref[...]

    c_ref[my] = jnp.dot(xbf, wdkv_ref[...].astype(BF),
                        preferred_element_type=F32).astype(BF)
    wukbf_ref[...] = wuk_ref[...].astype(BF)
    wuvbf_ref[...] = wuv_ref[...].astype(BF)
    wuk_sl[my] = wukbf_ref[:, pl.ds(my * DG, DG)]
    wuv_sl[my] = wuvbf_ref[:, pl.ds(my * DG, DG)]

    p1 = []
    for p in range(1, N_DEV):
        dst = lax.rem(my + p, N_DEV)
        for t, (src, dref) in enumerate((
                (c_ref.at[my], c_ref.at[my]),
                (wukbf_ref.at[:, pl.ds(dst * DG, DG)], wuk_sl.at[my]),
                (wuvbf_ref.at[:, pl.ds(dst * DG, DG)], wuv_sl.at[my]))):
            rdma = pltpu.make_async_remote_copy(
                src_ref=src, dst_ref=dref,
                send_sem=p1_send.at[p - 1, t],
                recv_sem=p1_recv.at[p - 1, t],
                device_id=(dst,),
                device_id_type=pl.DeviceIdType.MESH,
            )
            rdma.start()
            p1.append(rdma)

    kr = jnp.dot(xbf, wkr_ref[...].astype(BF),
                 preferred_element_type=F32).astype(BF)
    wqr_my = wqr_ref[:, pl.ds(my * HG * DR, HG * DR)].astype(BF)
    qr_my = jnp.dot(xbf, wqr_my, preferred_element_type=F32).astype(BF)
    wq_cp.wait()
    q_my = jnp.dot(xbf, wq_st[...].astype(BF),
                   preferred_element_type=F32).astype(BF)

    for rdma in p1:
        rdma.wait_recv()

    k_acc = jnp.zeros((BS, DG), F32)
    v_acc = jnp.zeros((BS, DG), F32)
    for o in range(N_DEV):
        k_acc = k_acc + jnp.dot(c_ref[o], wuk_sl[o],
                                preferred_element_type=F32)
        v_acc = v_acc + jnp.dot(c_ref[o], wuv_sl[o],
                                preferred_element_type=F32)
    k_my = k_acc.astype(BF)
    v_my = v_acc.astype(BF)

    scale = (DH + DR) ** -0.5
    nt = (((1,), (1,)), ((), ()))
    for hh in range(HG):
        ds_h = slice(hh * DH, (hh + 1) * DH)
        qh = jnp.concatenate([q_my[:, ds_h], qr_my[:, hh * DR:(hh + 1) * DR]],
                             axis=1)
        kh = jnp.concatenate([k_my[:, ds_h], kr], axis=1)
        vh = v_my[:, ds_h]
        for b in range(B):
            sl = slice(b * S, (b + 1) * S)
            s = lax.dot_general(qh[sl], kh[sl], nt,
                                preferred_element_type=F32)
            p = jnp.exp(s * scale)
            denom = jnp.sum(p, axis=-1, keepdims=True)
            o_b = jnp.dot(p.astype(BF), vh[sl], preferred_element_type=F32)
            obuf_ref[my, sl, ds_h] = (o_b * (1.0 / denom)).astype(BF)

    o_rdmas = []
    for p in range(1, N_DEV):
        dst = lax.rem(my + p, N_DEV)
        rdma = pltpu.make_async_remote_copy(
            src_ref=obuf_ref.at[my], dst_ref=obuf_ref.at[my],
            send_sem=o_send.at[p - 1], recv_sem=o_recv.at[p - 1],
            device_id=(dst,), device_id_type=pl.DeviceIdType.MESH,
        )
        rdma.start()
        o_rdmas.append(rdma)

    def proj(org, first):
        wo_cp.wait()
        wo_bf = wo_st[...].astype(BF)
        ob = obuf_ref[org]
        for j in range(2):
            half = pl.ds(j * (D // 2), D // 2)
            prod = jnp.dot(ob, wo_bf[:, j * (D // 2):(j + 1) * (D // 2)],
                           preferred_element_type=F32).reshape(B, S, D // 2)
            if first:
                out_ref[:, :, half] = prod
            else:
                out_ref[:, :, half] = out_ref[:, :, half] + prod

    proj(my, True)
    for p in range(1, N_DEV):
        org = lax.rem(my + N_DEV - p, N_DEV)
        cp = pltpu.make_async_copy(
            wo_any.at[pl.ds(org * DG, DG), :], wo_st, wo_sem)
        cp.start()
        o_rdmas[p - 1].wait_recv()
        proj(org, False)

    for rdma in p1 + o_rdmas:
        rdma.wait_send()


def kernel(x, Wdkv, Wuk, Wuv, Wq, Wqr, Wkr, Wo):
    return pl.pallas_call(
        _fused_body,
        out_shape=jax.ShapeDtypeStruct((B, S, D), F32),
        in_specs=[pl.BlockSpec(memory_space=pltpu.VMEM)] * 6
        + [pl.BlockSpec(memory_space=pl.ANY)] * 2,
        out_specs=pl.BlockSpec(memory_space=pltpu.VMEM),
        scratch_shapes=[
            pltpu.VMEM((BS, D), BF),
            pltpu.VMEM((N_DEV, BS, DCS), BF),
            pltpu.VMEM((DCS, D), BF),
            pltpu.VMEM((DCS, D), BF),
            pltpu.VMEM((N_DEV, DCS, DG), BF),
            pltpu.VMEM((N_DEV, DCS, DG), BF),
            pltpu.VMEM((N_DEV, BS, DG), BF),
            pltpu.VMEM((D, DG), F32),
            pltpu.VMEM((DG, D), F32),
            pltpu.SemaphoreType.DMA((N_DEV - 1, N_COMM)),
            pltpu.SemaphoreType.DMA((N_DEV - 1, N_COMM)),
            pltpu.SemaphoreType.DMA((N_DEV - 1,)),
            pltpu.SemaphoreType.DMA((N_DEV - 1,)),
            pltpu.SemaphoreType.DMA,
            pltpu.SemaphoreType.DMA,
        ],
    )(x, Wdkv, Wuk, Wuv, Wkr, Wqr, Wq, Wo)


# device time: 69776 ns/iter; 1.4274x vs baseline; 1.0491x over previous
import jax
import jax.numpy as jnp
from jax import lax
from jax.experimental import pallas as pl
from jax.experimental.pallas import tpu as pltpu

N_DEV = 4
B, S, D = 2, 512, 2048
H, DH, DR = 16, 128, 32
DC = 512
DCS = DC // N_DEV
BS = B * S
HG = H // N_DEV
DG = HG * DH
N_COMM = 3

BF = jnp.bfloat16
F32 = jnp.float32


def _fused_body(x_ref, wdkv_ref, wuk_ref, wuv_ref, wkr_ref, wqr_ref,
                wq_any, wo_any, out_ref,
                xbf_ref, c_ref, wukbf_ref, wuvbf_ref, wuk_sl, wuv_sl,
                obuf_ref, wq_st, wo_st,
                p1_send, p1_recv, o_send, o_recv, wq_sem, wo_sem):
    my = lax.axis_index("i")

    barrier = pltpu.get_barrier_semaphore()
    for p in range(1, N_DEV):
        pl.semaphore_signal(barrier, inc=1,
                            device_id=(lax.rem(my + p, N_DEV),),
                            device_id_type=pl.DeviceIdType.MESH)

    wq_cp = pltpu.make_async_copy(
        wq_any.at[:, pl.ds(my * DG, DG)], wq_st, wq_sem)
    wq_cp.start()
    wo_cp = pltpu.make_async_copy(
        wo_any.at[pl.ds(my * DG, DG), :], wo_st, wo_sem)
    wo_cp.start()

    for b in range(B):
        xbf_ref[b * S:(b + 1) * S, :] = x_ref[b].astype(BF)
    xbf = xbf_ref[...]

    c_ref[my] = jnp.dot(xbf, wdkv_ref[...].astype(BF),
                        preferred_element_type=F32).astype(BF)
    wukbf_ref[...] = wuk_ref[...].astype(BF)
    wuvbf_ref[...] = wuv_ref[...].astype(BF)
    wuk_sl[my] = wukbf_ref[:, pl.ds(my * DG, DG)]
    wuv_sl[my] = wuvbf_ref[:, pl.ds(my * DG, DG)]

    pl.semaphore_wait(barrier, N_DEV - 1)
    p1 = []
    for p in range(1, N_DEV):
        dst = lax.rem(my + p, N_DEV)
        for t, (src, dref) in enumerate((
                (c_ref.at[my], c_ref.at[my]),
                (wukbf_ref.at[:, pl.ds(dst * DG, DG)], wuk_sl.at[my]),
                (wuvbf_ref.at[:, pl.ds(dst * DG, DG)], wuv_sl.at[my]))):
            rdma = pltpu.make_async_remote_copy(
                src_ref=src, dst_ref=dref,
                send_sem=p1_send.at[p - 1, t],
                recv_sem=p1_recv.at[p - 1, t],
                device_id=(dst,),
                device_id_type=pl.DeviceIdType.MESH,
            )
            rdma.start()
            p1.append(rdma)

    kr = jnp.dot(xbf, wkr_ref[...].astype(BF),
                 preferred_element_type=F32).astype(BF)
    wqr_my = wqr_ref[:, pl.ds(my * HG * DR, HG * DR)].astype(BF)
    qr_my = jnp.dot(xbf, wqr_my, preferred_element_type=F32).astype(BF)
    wq_cp.wait()
    q_my = jnp.dot(xbf, wq_st[...].astype(BF),
                   preferred_element_type=F32).astype(BF)

    for rdma in p1:
        rdma.wait_recv()

    k_acc = jnp.zeros((BS, DG), F32)
    v_acc = jnp.zeros((BS, DG), F32)
    for o in range(N_DEV):
        k_acc = k_acc + jnp.dot(c_ref[o], wuk_sl[o],
                                preferred_element_type=F32)
        v_acc = v_acc + jnp.dot(c_ref[o], wuv_sl[o],
                                preferred_element_type=F32)
    k_my = k_acc.astype(BF)
    v_my = v_acc.astype(BF)

    scale = (DH + DR) ** -0.5
    nt = (((1,), (1,)), ((), ()))
    for hh in range(HG):
        ds_h = slice(hh * DH, (hh + 1) * DH)
        qh = jnp.concatenate([q_my[:, ds_h], qr_my[:, hh * DR:(hh + 1) * DR]],
                             axis=1)
        kh = jnp.concatenate([k_my[:, ds_h], kr], axis=1)
        vh = v_my[:, ds_h]
        for b in range(B):
            sl = slice(b * S, (b + 1) * S)
            s = lax.dot_general(qh[sl], kh[sl], nt,
                                preferred_element_type=F32)
            p = jnp.exp(s * scale)
            denom = jnp.sum(p, axis=-1, keepdims=True)
            o_b = jnp.dot(p.astype(BF), vh[sl], preferred_element_type=F32)
            obuf_ref[my, sl, ds_h] = (o_b * (1.0 / denom)).astype(BF)

    o_rdmas = []
    for p in range(1, N_DEV):
        dst = lax.rem(my + p, N_DEV)
        rdma = pltpu.make_async_remote_copy(
            src_ref=obuf_ref.at[my], dst_ref=obuf_ref.at[my],
            send_sem=o_send.at[p - 1], recv_sem=o_recv.at[p - 1],
            device_id=(dst,), device_id_type=pl.DeviceIdType.MESH,
        )
        rdma.start()
        o_rdmas.append(rdma)

    def proj(org, first):
        wo_cp.wait()
        wo_bf = wo_st[...].astype(BF)
        ob = obuf_ref[org]
        for j in range(2):
            half = pl.ds(j * (D // 2), D // 2)
            prod = jnp.dot(ob, wo_bf[:, j * (D // 2):(j + 1) * (D // 2)],
                           preferred_element_type=F32).reshape(B, S, D // 2)
            if first:
                out_ref[:, :, half] = prod
            else:
                out_ref[:, :, half] = out_ref[:, :, half] + prod

    proj(my, True)
    for p in range(1, N_DEV):
        org = lax.rem(my + N_DEV - p, N_DEV)
        cp = pltpu.make_async_copy(
            wo_any.at[pl.ds(org * DG, DG), :], wo_st, wo_sem)
        cp.start()
        o_rdmas[p - 1].wait_recv()
        proj(org, False)

    for rdma in p1 + o_rdmas:
        rdma.wait_send()


def kernel(x, Wdkv, Wuk, Wuv, Wq, Wqr, Wkr, Wo):
    return pl.pallas_call(
        _fused_body,
        out_shape=jax.ShapeDtypeStruct((B, S, D), F32),
        in_specs=[pl.BlockSpec(memory_space=pltpu.VMEM)] * 6
        + [pl.BlockSpec(memory_space=pl.ANY)] * 2,
        out_specs=pl.BlockSpec(memory_space=pltpu.VMEM),
        scratch_shapes=[
            pltpu.VMEM((BS, D), BF),
            pltpu.VMEM((N_DEV, BS, DCS), BF),
            pltpu.VMEM((DCS, D), BF),
            pltpu.VMEM((DCS, D), BF),
            pltpu.VMEM((N_DEV, DCS, DG), BF),
            pltpu.VMEM((N_DEV, DCS, DG), BF),
            pltpu.VMEM((N_DEV, BS, DG), BF),
            pltpu.VMEM((D, DG), F32),
            pltpu.VMEM((DG, D), F32),
            pltpu.SemaphoreType.DMA((N_DEV - 1, N_COMM)),
            pltpu.SemaphoreType.DMA((N_DEV - 1, N_COMM)),
            pltpu.SemaphoreType.DMA((N_DEV - 1,)),
            pltpu.SemaphoreType.DMA((N_DEV - 1,)),
            pltpu.SemaphoreType.DMA,
            pltpu.SemaphoreType.DMA,
        ],
        compiler_params=pltpu.CompilerParams(collective_id=0),
    )(x, Wdkv, Wuk, Wuv, Wkr, Wqr, Wq, Wo)


# device time: 69540 ns/iter; 1.4323x vs baseline; 1.0034x over previous
import jax
import jax.numpy as jnp
from jax import lax
from jax.experimental import pallas as pl
from jax.experimental.pallas import tpu as pltpu

N_DEV = 4
B, S, D = 2, 512, 2048
H, DH, DR = 16, 128, 32
DC = 512
DCS = DC // N_DEV
BS = B * S
HG = H // N_DEV
DG = HG * DH
N_COMM = 3

BF = jnp.bfloat16
F32 = jnp.float32


def _fused_body(x_ref, wdkv_ref, wuk_ref, wuv_ref, wkr_ref, wqr_ref,
                wq_any, wo_any, out_ref,
                xbf_ref, c_ref, wukbf_ref, wuvbf_ref, wuk_sl, wuv_sl,
                obuf_ref, wq_st, wo_st,
                p1_send, p1_recv, o_send, o_recv, wq_sem, wo_sem):
    my = lax.axis_index("i")

    barrier = pltpu.get_barrier_semaphore()
    for p in range(1, N_DEV):
        pl.semaphore_signal(barrier, inc=1,
                            device_id=(lax.rem(my + p, N_DEV),),
                            device_id_type=pl.DeviceIdType.MESH)

    wq_cp = pltpu.make_async_copy(
        wq_any.at[:, pl.ds(my * DG, DG)], wq_st, wq_sem)
    wq_cp.start()
    wo_cp = pltpu.make_async_copy(
        wo_any.at[pl.ds(my * DG, DG), :], wo_st, wo_sem)
    wo_cp.start()

    for b in range(B):
        xbf_ref[b * S:(b + 1) * S, :] = x_ref[b].astype(BF)
    xbf = xbf_ref[...]

    c_ref[my] = jnp.dot(xbf, wdkv_ref[...].astype(BF),
                        preferred_element_type=F32).astype(BF)
    wukbf_ref[...] = wuk_ref[...].astype(BF)
    wuvbf_ref[...] = wuv_ref[...].astype(BF)
    wuk_sl[my] = wukbf_ref[:, pl.ds(my * DG, DG)]
    wuv_sl[my] = wuvbf_ref[:, pl.ds(my * DG, DG)]

    pl.semaphore_wait(barrier, N_DEV - 1)
    p1 = []
    for p in range(1, N_DEV):
        dst = lax.rem(my + p, N_DEV)
        for t, (src, dref) in enumerate((
                (c_ref.at[my], c_ref.at[my]),
                (wukbf_ref.at[:, pl.ds(dst * DG, DG)], wuk_sl.at[my]),
                (wuvbf_ref.at[:, pl.ds(dst * DG, DG)], wuv_sl.at[my]))):
            rdma = pltpu.make_async_remote_copy(
                src_ref=src, dst_ref=dref,
                send_sem=p1_send.at[p - 1, t],
                recv_sem=p1_recv.at[p - 1, t],
                device_id=(dst,),
                device_id_type=pl.DeviceIdType.MESH,
            )
            rdma.start()
            p1.append(rdma)

    kr = jnp.dot(xbf, wkr_ref[...].astype(BF),
                 preferred_element_type=F32).astype(BF)
    wqr_my = wqr_ref[:, pl.ds(my * HG * DR, HG * DR)].astype(BF)
    qr_my = jnp.dot(xbf, wqr_my, preferred_element_type=F32).astype(BF)
    wq_cp.wait()
    q_my = jnp.dot(xbf, wq_st[...].astype(BF),
                   preferred_element_type=F32).astype(BF)

    k_acc = jnp.dot(c_ref[my], wuk_sl[my], preferred_element_type=F32)
    v_acc = jnp.dot(c_ref[my], wuv_sl[my], preferred_element_type=F32)
    for p in range(1, N_DEV):
        for t in range(N_COMM):
            p1[(p - 1) * N_COMM + t].wait_recv()
        org = lax.rem(my + N_DEV - p, N_DEV)
        k_acc = k_acc + jnp.dot(c_ref[org], wuk_sl[org],
                                preferred_element_type=F32)
        v_acc = v_acc + jnp.dot(c_ref[org], wuv_sl[org],
                                preferred_element_type=F32)
    k_my = k_acc.astype(BF)
    v_my = v_acc.astype(BF)

    scale = (DH + DR) ** -0.5
    nt = (((1,), (1,)), ((), ()))
    o_rdmas = []
    for hh in range(HG):
        ds_h = slice(hh * DH, (hh + 1) * DH)
        qh = jnp.concatenate([q_my[:, ds_h], qr_my[:, hh * DR:(hh + 1) * DR]],
                             axis=1)
        kh = jnp.concatenate([k_my[:, ds_h], kr], axis=1)
        vh = v_my[:, ds_h]
        for b in range(B):
            sl = slice(b * S, (b + 1) * S)
            s = lax.dot_general(qh[sl], kh[sl], nt,
                                preferred_element_type=F32)
            p = jnp.exp(s * scale)
            denom = jnp.sum(p, axis=-1, keepdims=True)
            o_b = jnp.dot(p.astype(BF), vh[sl], preferred_element_type=F32)
            obuf_ref[my, sl, ds_h] = (o_b * (1.0 / denom)).astype(BF)
        stripe = pl.ds(hh * DH, DH)
        for p in range(1, N_DEV):
            dst = lax.rem(my + p, N_DEV)
            rdma = pltpu.make_async_remote_copy(
                src_ref=obuf_ref.at[my, :, stripe],
                dst_ref=obuf_ref.at[my, :, stripe],
                send_sem=o_send.at[p - 1, hh], recv_sem=o_recv.at[p - 1, hh],
                device_id=(dst,), device_id_type=pl.DeviceIdType.MESH,
            )
            rdma.start()
            o_rdmas.append(rdma)

    def proj(org, first):
        wo_cp.wait()
        wo_bf = wo_st[...].astype(BF)
        ob = obuf_ref[org]
        for j in range(2):
            half = pl.ds(j * (D // 2), D // 2)
            prod = jnp.dot(ob, wo_bf[:, j * (D // 2):(j + 1) * (D // 2)],
                           preferred_element_type=F32).reshape(B, S, D // 2)
            if first:
                out_ref[:, :, half] = prod
            else:
                out_ref[:, :, half] = out_ref[:, :, half] + prod

    proj(my, True)
    for p in range(1, N_DEV):
        org = lax.rem(my + N_DEV - p, N_DEV)
        cp = pltpu.make_async_copy(
            wo_any.at[pl.ds(org * DG, DG), :], wo_st, wo_sem)
        cp.start()
        for hh in range(HG):
            o_rdmas[hh * (N_DEV - 1) + (p - 1)].wait_recv()
        proj(org, False)

    for rdma in p1 + o_rdmas:
        rdma.wait_send()


def kernel(x, Wdkv, Wuk, Wuv, Wq, Wqr, Wkr, Wo):
    return pl.pallas_call(
        _fused_body,
        out_shape=jax.ShapeDtypeStruct((B, S, D), F32),
        in_specs=[pl.BlockSpec(memory_space=pltpu.VMEM)] * 6
        + [pl.BlockSpec(memory_space=pl.ANY)] * 2,
        out_specs=pl.BlockSpec(memory_space=pltpu.VMEM),
        scratch_shapes=[
            pltpu.VMEM((BS, D), BF),
            pltpu.VMEM((N_DEV, BS, DCS), BF),
            pltpu.VMEM((DCS, D), BF),
            pltpu.VMEM((DCS, D), BF),
            pltpu.VMEM((N_DEV, DCS, DG), BF),
            pltpu.VMEM((N_DEV, DCS, DG), BF),
            pltpu.VMEM((N_DEV, BS, DG), BF),
            pltpu.VMEM((D, DG), F32),
            pltpu.VMEM((DG, D), F32),
            pltpu.SemaphoreType.DMA((N_DEV - 1, N_COMM)),
            pltpu.SemaphoreType.DMA((N_DEV - 1, N_COMM)),
            pltpu.SemaphoreType.DMA((N_DEV - 1, HG)),
            pltpu.SemaphoreType.DMA((N_DEV - 1, HG)),
            pltpu.SemaphoreType.DMA,
            pltpu.SemaphoreType.DMA,
        ],
        compiler_params=pltpu.CompilerParams(collective_id=0),
    )(x, Wdkv, Wuk, Wuv, Wkr, Wqr, Wq, Wo)
